# Initial kernel scaffold; baseline (speedup 1.0000x reference)
import jax
import jax.numpy as jnp
from jax.experimental import pallas as pl


def kernel(x, edge_index, lin1_W, lin1_b, ln_g, ln_b, proj_W, proj_b, ll_W, ll_b, lr_W, norm_g, norm_b, lin2_W, lin2_b):
    raise NotImplementedError("write your pallas kernel here")


# speedup vs baseline: 4.6094x; 4.6094x over previous
import functools

import jax
import jax.numpy as jnp
from jax import lax
from jax.experimental import pallas as pl
from jax.experimental.pallas import tpu as pltpu
from jax.experimental.pallas import tpu_sc as plsc

N = 10000
E = 320000
D_IN = 128
HID = 128
OUT = 64
L = 3
G = 2
C = HID // G

NC = 2
NS = 16
NW = NC * NS
EPW = E // NW
CHUNK = 80
NCHUNK = EPW // CHUNK
N_PAD = 10240
RPT = N_PAD // NS
DEGW = 8


def _segsum_body(with_deg, *refs):
    if with_deg:
        (p_hbm, src_hbm, dst_hbm, z64_hbm, z8_hbm, ones_hbm,
         agg_out, deg_out, src_v, dst_v, rows_v, ones_v, acc_sh, dacc_sh) = refs
    else:
        (p_hbm, src_hbm, dst_hbm, z64_hbm,
         agg_out, src_v, dst_v, rows_v, acc_sh) = refs

    c = lax.axis_index("c")
    s = lax.axis_index("s")
    w = c * NS + s
    r0 = s * RPT

    pltpu.sync_copy(z64_hbm, acc_sh.at[pl.ds(r0, RPT)])
    if with_deg:
        pltpu.sync_copy(z8_hbm, dacc_sh.at[pl.ds(r0, RPT)])
        pltpu.sync_copy(ones_hbm, ones_v)
    plsc.subcore_barrier()

    ebase = w * EPW

    def step(j, carry):
        b = ebase + j * CHUNK
        pltpu.sync_copy(src_hbm.at[pl.ds(b, CHUNK)], src_v)
        pltpu.sync_copy(dst_hbm.at[pl.ds(b, CHUNK)], dst_v)
        pltpu.sync_copy(p_hbm.at[src_v], rows_v)
        pltpu.sync_copy(rows_v, acc_sh.at[dst_v], add=True)
        if with_deg:
            pltpu.sync_copy(ones_v, dacc_sh.at[dst_v], add=True)
        return carry

    lax.fori_loop(0, NCHUNK, step, 0)
    plsc.subcore_barrier()

    pltpu.sync_copy(acc_sh.at[pl.ds(r0, RPT)], agg_out.at[c, pl.ds(r0, RPT)])
    if with_deg:
        pltpu.sync_copy(dacc_sh.at[pl.ds(r0, RPT)],
                        deg_out.at[c, pl.ds(r0, RPT)])


def _make_segsum(with_deg):
    mesh = plsc.VectorSubcoreMesh(core_axis_name="c", subcore_axis_name="s")
    out_type = [jax.ShapeDtypeStruct((NC, N_PAD, C), jnp.float32)]
    scratch = [
        pltpu.VMEM((CHUNK,), jnp.int32),
        pltpu.VMEM((CHUNK,), jnp.int32),
        pltpu.VMEM((CHUNK, C), jnp.float32),
    ]
    if with_deg:
        out_type.append(jax.ShapeDtypeStruct((NC, N_PAD, DEGW), jnp.float32))
        scratch.append(pltpu.VMEM((CHUNK, DEGW), jnp.float32))
        scratch.append(pltpu.VMEM_SHARED((N_PAD, C), jnp.float32))
        scratch.append(pltpu.VMEM_SHARED((N_PAD, DEGW), jnp.float32))
    else:
        scratch.append(pltpu.VMEM_SHARED((N_PAD, C), jnp.float32))
    return pl.kernel(
        functools.partial(_segsum_body, with_deg),
        out_type=out_type,
        mesh=mesh,
        scratch_types=scratch,
        compiler_params=pltpu.CompilerParams(use_tc_tiling_on_sc=False),
    )


RB = 1000
GRID = N // RB

def _ln(h, g, b):
    mu = jnp.mean(h, axis=-1, keepdims=True)
    var = jnp.mean((h - mu) ** 2, axis=-1, keepdims=True)
    return (h - mu) / jnp.sqrt(var + 1e-5) * g + b


def _lin1_body(x_ref, w_ref, b_ref, o0_ref, o1_ref):
    h = jnp.dot(x_ref[...], w_ref[...],
                preferred_element_type=jnp.float32) + b_ref[...]
    o0_ref[...] = h[:, :C]
    o1_ref[...] = h[:, C:]


def _lin1(x, w, b):
    return pl.pallas_call(
        _lin1_body,
        grid=(GRID,),
        in_specs=[
            pl.BlockSpec((RB, D_IN), lambda i: (i, 0)),
            pl.BlockSpec((D_IN, HID), lambda i: (0, 0)),
            pl.BlockSpec((1, HID), lambda i: (0, 0)),
        ],
        out_specs=[
            pl.BlockSpec((RB, C), lambda i: (i, 0)),
            pl.BlockSpec((RB, C), lambda i: (i, 0)),
        ],
        out_shape=[
            jax.ShapeDtypeStruct((N, C), jnp.float32),
            jax.ShapeDtypeStruct((N, C), jnp.float32),
        ],
    )(x, w, b)


def _pre_body(y_ref, lg_ref, lb_ref, pw_ref, pb_ref, z_ref, p_ref):
    z = jax.nn.relu(_ln(y_ref[...], lg_ref[...], lb_ref[...]))
    z_ref[...] = z
    p_ref[...] = jax.nn.relu(
        jnp.dot(z, pw_ref[...], preferred_element_type=jnp.float32)
        + pb_ref[...])


def _pre(y, lg, lb, pw, pb):
    return pl.pallas_call(
        _pre_body,
        grid=(GRID,),
        in_specs=[
            pl.BlockSpec((RB, C), lambda i: (i, 0)),
            pl.BlockSpec((1, C), lambda i: (0, 0)),
            pl.BlockSpec((1, C), lambda i: (0, 0)),
            pl.BlockSpec((C, C), lambda i: (0, 0)),
            pl.BlockSpec((1, C), lambda i: (0, 0)),
        ],
        out_specs=[
            pl.BlockSpec((RB, C), lambda i: (i, 0)),
            pl.BlockSpec((RB, C), lambda i: (i, 0)),
        ],
        out_shape=[
            jax.ShapeDtypeStruct((N, C), jnp.float32),
            jax.ShapeDtypeStruct((N, C), jnp.float32),
        ],
    )(y, lg, lb, pw, pb)


def _post_body(a0_ref, a1_ref, d0_ref, d1_ref, res_ref, z_ref,
               lw_ref, lb_ref, rw_ref, y_ref):
    a0 = a0_ref[0]
    a1 = a1_ref[0]
    d0 = d0_ref[0]
    d1 = d1_ref[0]
    deg = jnp.clip(d0[:, :1] + d1[:, :1], 1.0, None)
    agg = (a0 + a1) / deg
    y_ref[...] = (res_ref[...]
                  + jnp.dot(agg, lw_ref[...],
                            preferred_element_type=jnp.float32)
                  + lb_ref[...]
                  + jnp.dot(z_ref[...], rw_ref[...],
                            preferred_element_type=jnp.float32))


def _post(agg2, deg2, res, z, lw, lb, rw):
    return pl.pallas_call(
        _post_body,
        grid=(GRID,),
        in_specs=[
            pl.BlockSpec((1, RB, C), lambda i: (0, i, 0)),
            pl.BlockSpec((1, RB, C), lambda i: (1, i, 0)),
            pl.BlockSpec((1, RB, DEGW), lambda i: (0, i, 0)),
            pl.BlockSpec((1, RB, DEGW), lambda i: (1, i, 0)),
            pl.BlockSpec((RB, C), lambda i: (i, 0)),
            pl.BlockSpec((RB, C), lambda i: (i, 0)),
            pl.BlockSpec((C, C), lambda i: (0, 0)),
            pl.BlockSpec((1, C), lambda i: (0, 0)),
            pl.BlockSpec((C, C), lambda i: (0, 0)),
        ],
        out_specs=pl.BlockSpec((RB, C), lambda i: (i, 0)),
        out_shape=jax.ShapeDtypeStruct((N, C), jnp.float32),
    )(agg2, agg2, deg2, deg2, res, z, lw, lb, rw)


def _final_body(y0_ref, y1_ref, g_ref, b_ref, w_ref, b2_ref, o_ref):
    h = jnp.concatenate([y0_ref[...], y1_ref[...]], axis=1)
    h = jax.nn.relu(_ln(h, g_ref[...], b_ref[...]))
    o_ref[...] = jnp.dot(h, w_ref[...],
                         preferred_element_type=jnp.float32) + b2_ref[...]


def _final(y0, y1, g, b, w, b2):
    return pl.pallas_call(
        _final_body,
        grid=(GRID,),
        in_specs=[
            pl.BlockSpec((RB, C), lambda i: (i, 0)),
            pl.BlockSpec((RB, C), lambda i: (i, 0)),
            pl.BlockSpec((1, HID), lambda i: (0, 0)),
            pl.BlockSpec((1, HID), lambda i: (0, 0)),
            pl.BlockSpec((HID, OUT), lambda i: (0, 0)),
            pl.BlockSpec((1, OUT), lambda i: (0, 0)),
        ],
        out_specs=pl.BlockSpec((RB, OUT), lambda i: (i, 0)),
        out_shape=jax.ShapeDtypeStruct((N, OUT), jnp.float32),
    )(y0, y1, g, b, w, b2)


def kernel(x, edge_index, lin1_W, lin1_b, ln_g, ln_b, proj_W, proj_b,
           ll_W, ll_b, lr_W, norm_g, norm_b, lin2_W, lin2_b):
    src = edge_index[0]
    dst = edge_index[1]

    z64 = jnp.zeros((RPT, C), jnp.float32)
    z8 = jnp.zeros((RPT, DEGW), jnp.float32)
    ones = jnp.ones((CHUNK, DEGW), jnp.float32)

    seg_deg = _make_segsum(True)
    seg = _make_segsum(False)

    y0, y1 = _lin1(x, lin1_W, lin1_b.reshape(1, HID))

    deg2 = None
    for i in range(L):
        for g in range(G):
            y_in = y1 if g == 0 else y0
            res = y0 if g == 0 else y1
            z, p = _pre(y_in,
                        ln_g[i, g].reshape(1, C), ln_b[i, g].reshape(1, C),
                        proj_W[i, g], proj_b[i, g].reshape(1, C))
            if deg2 is None:
                agg2, deg2 = seg_deg(p, src, dst, z64, z8, ones)
            else:
                (agg2,) = seg(p, src, dst, z64)
            y_new = _post(agg2, deg2, res, z,
                          ll_W[i, g], ll_b[i, g].reshape(1, C), lr_W[i, g])
            if g == 0:
                y0 = y_new
            else:
                y1 = y_new

    return _final(y0, y1, norm_g.reshape(1, HID), norm_b.reshape(1, HID),
                  lin2_W, lin2_b.reshape(1, OUT))


# speedup vs baseline: 10.5934x; 2.2982x over previous
import functools

import jax
import jax.numpy as jnp
from jax import lax
from jax.experimental import pallas as pl
from jax.experimental.pallas import tpu as pltpu
from jax.experimental.pallas import tpu_sc as plsc

N = 10000
E = 320000
D_IN = 128
HID = 128
OUT = 64
L = 3
G = 2
C = HID // G

NC = 2
NS = 16
NW = NC * NS
EPW = E // NW
CHUNK = 100
NCHUNK = EPW // CHUNK
NBUF = 5
NGROUP = NCHUNK // NBUF
N_PAD = 10240
RPT = N_PAD // NS
WDEG = 80


def _segsum_body(width, p_hbm, src_hbm, dst_hbm, z_hbm, agg_out,
                 sidx_v, didx_v, bufs, sems_g, sem_s, acc_sh):
    c = lax.axis_index("c")
    s = lax.axis_index("s")
    w = c * NS + s
    r0 = s * RPT

    pltpu.sync_copy(src_hbm.at[w], sidx_v)
    pltpu.sync_copy(dst_hbm.at[w], didx_v)
    pltpu.sync_copy(z_hbm, acc_sh.at[pl.ds(r0, RPT)])
    plsc.subcore_barrier()

    def group(g, carry):
        base = g * NBUF
        gh = [pltpu.async_copy(p_hbm.at[sidx_v.at[base + b]],
                               bufs[b], sems_g[b])
              for b in range(NBUF)]
        sh = []
        for b in range(NBUF):
            gh[b].wait()
            sh.append(pltpu.async_copy(bufs[b],
                                       acc_sh.at[didx_v.at[base + b]],
                                       sem_s, add=True))
        for h in sh:
            h.wait()
        return carry

    lax.fori_loop(0, NGROUP, group, 0)
    plsc.subcore_barrier()

    pltpu.sync_copy(acc_sh.at[pl.ds(r0, RPT)], agg_out.at[c, pl.ds(r0, RPT)])


def _make_segsum(width):
    mesh = plsc.VectorSubcoreMesh(core_axis_name="c", subcore_axis_name="s")

    def body(p_hbm, src_hbm, dst_hbm, z_hbm, agg_out,
             sidx_v, didx_v, *rest):
        bufs = rest[:NBUF]
        sems_g = rest[NBUF:2 * NBUF]
        sem_s = rest[2 * NBUF]
        acc_sh = rest[2 * NBUF + 1]
        _segsum_body(width, p_hbm, src_hbm, dst_hbm, z_hbm, agg_out,
                     sidx_v, didx_v, bufs, sems_g, sem_s, acc_sh)

    scratch = [
        pltpu.VMEM((NCHUNK, CHUNK), jnp.int32),
        pltpu.VMEM((NCHUNK, CHUNK), jnp.int32),
    ]
    scratch += [pltpu.VMEM((CHUNK, width), jnp.float32) for _ in range(NBUF)]
    scratch += [pltpu.SemaphoreType.DMA for _ in range(NBUF + 1)]
    scratch.append(pltpu.VMEM_SHARED((N_PAD, width), jnp.float32))

    return pl.kernel(
        body,
        out_type=jax.ShapeDtypeStruct((NC, N_PAD, width), jnp.float32),
        mesh=mesh,
        scratch_types=scratch,
        compiler_params=pltpu.CompilerParams(use_tc_tiling_on_sc=False),
    )


RB = 1000
GRID = N // RB

def _ln(h, g, b):
    mu = jnp.mean(h, axis=-1, keepdims=True)
    var = jnp.mean((h - mu) ** 2, axis=-1, keepdims=True)
    return (h - mu) / jnp.sqrt(var + 1e-5) * g + b


def _lin1_body(x_ref, w_ref, b_ref, o0_ref, o1_ref):
    h = jnp.dot(x_ref[...], w_ref[...],
                preferred_element_type=jnp.float32) + b_ref[...]
    o0_ref[...] = h[:, :C]
    o1_ref[...] = h[:, C:]


def _lin1(x, w, b):
    return pl.pallas_call(
        _lin1_body,
        grid=(GRID,),
        in_specs=[
            pl.BlockSpec((RB, D_IN), lambda i: (i, 0)),
            pl.BlockSpec((D_IN, HID), lambda i: (0, 0)),
            pl.BlockSpec((1, HID), lambda i: (0, 0)),
        ],
        out_specs=[
            pl.BlockSpec((RB, C), lambda i: (i, 0)),
            pl.BlockSpec((RB, C), lambda i: (i, 0)),
        ],
        out_shape=[
            jax.ShapeDtypeStruct((N, C), jnp.float32),
            jax.ShapeDtypeStruct((N, C), jnp.float32),
        ],
    )(x, w, b)


def _pre_body(wout, y_ref, lg_ref, lb_ref, pw_ref, pb_ref, z_ref, p_ref):
    z = jax.nn.relu(_ln(y_ref[...], lg_ref[...], lb_ref[...]))
    z_ref[...] = z
    p = jax.nn.relu(
        jnp.dot(z, pw_ref[...], preferred_element_type=jnp.float32)
        + pb_ref[...])
    if wout > C:
        p = jnp.concatenate(
            [p, jnp.ones((p.shape[0], wout - C), jnp.float32)], axis=1)
    p_ref[...] = p


def _pre(y, lg, lb, pw, pb, wout):
    return pl.pallas_call(
        functools.partial(_pre_body, wout),
        grid=(GRID,),
        in_specs=[
            pl.BlockSpec((RB, C), lambda i: (i, 0)),
            pl.BlockSpec((1, C), lambda i: (0, 0)),
            pl.BlockSpec((1, C), lambda i: (0, 0)),
            pl.BlockSpec((C, C), lambda i: (0, 0)),
            pl.BlockSpec((1, C), lambda i: (0, 0)),
        ],
        out_specs=[
            pl.BlockSpec((RB, C), lambda i: (i, 0)),
            pl.BlockSpec((RB, wout), lambda i: (i, 0)),
        ],
        out_shape=[
            jax.ShapeDtypeStruct((N, C), jnp.float32),
            jax.ShapeDtypeStruct((N, wout), jnp.float32),
        ],
    )(y, lg, lb, pw, pb)


def _post_body(wa, a0_ref, a1_ref, d0_ref, d1_ref, res_ref, z_ref,
               lw_ref, lb_ref, rw_ref, y_ref):
    deg = jnp.clip(d0_ref[0][:, C:C + 1] + d1_ref[0][:, C:C + 1], 1.0, None)
    agg = (a0_ref[0][:, :C] + a1_ref[0][:, :C]) / deg
    y_ref[...] = (res_ref[...]
                  + jnp.dot(agg, lw_ref[...],
                            preferred_element_type=jnp.float32)
                  + lb_ref[...]
                  + jnp.dot(z_ref[...], rw_ref[...],
                            preferred_element_type=jnp.float32))


def _post(agg2, deg2, res, z, lw, lb, rw):
    wa = agg2.shape[-1]
    return pl.pallas_call(
        functools.partial(_post_body, wa),
        grid=(GRID,),
        in_specs=[
            pl.BlockSpec((1, RB, wa), lambda i: (0, i, 0)),
            pl.BlockSpec((1, RB, wa), lambda i: (1, i, 0)),
            pl.BlockSpec((1, RB, WDEG), lambda i: (0, i, 0)),
            pl.BlockSpec((1, RB, WDEG), lambda i: (1, i, 0)),
            pl.BlockSpec((RB, C), lambda i: (i, 0)),
            pl.BlockSpec((RB, C), lambda i: (i, 0)),
            pl.BlockSpec((C, C), lambda i: (0, 0)),
            pl.BlockSpec((1, C), lambda i: (0, 0)),
            pl.BlockSpec((C, C), lambda i: (0, 0)),
        ],
        out_specs=pl.BlockSpec((RB, C), lambda i: (i, 0)),
        out_shape=jax.ShapeDtypeStruct((N, C), jnp.float32),
    )(agg2, agg2, deg2, deg2, res, z, lw, lb, rw)


def _final_body(y0_ref, y1_ref, g_ref, b_ref, w_ref, b2_ref, o_ref):
    h = jnp.concatenate([y0_ref[...], y1_ref[...]], axis=1)
    h = jax.nn.relu(_ln(h, g_ref[...], b_ref[...]))
    o_ref[...] = jnp.dot(h, w_ref[...],
                         preferred_element_type=jnp.float32) + b2_ref[...]


def _final(y0, y1, g, b, w, b2):
    return pl.pallas_call(
        _final_body,
        grid=(GRID,),
        in_specs=[
            pl.BlockSpec((RB, C), lambda i: (i, 0)),
            pl.BlockSpec((RB, C), lambda i: (i, 0)),
            pl.BlockSpec((1, HID), lambda i: (0, 0)),
            pl.BlockSpec((1, HID), lambda i: (0, 0)),
            pl.BlockSpec((HID, OUT), lambda i: (0, 0)),
            pl.BlockSpec((1, OUT), lambda i: (0, 0)),
        ],
        out_specs=pl.BlockSpec((RB, OUT), lambda i: (i, 0)),
        out_shape=jax.ShapeDtypeStruct((N, OUT), jnp.float32),
    )(y0, y1, g, b, w, b2)


def kernel(x, edge_index, lin1_W, lin1_b, ln_g, ln_b, proj_W, proj_b,
           ll_W, ll_b, lr_W, norm_g, norm_b, lin2_W, lin2_b):
    src3 = edge_index[0].reshape(NW, NCHUNK, CHUNK)
    dst3 = edge_index[1].reshape(NW, NCHUNK, CHUNK)

    zdeg = jnp.zeros((RPT, WDEG), jnp.float32)
    z64 = jnp.zeros((RPT, C), jnp.float32)

    seg_deg = _make_segsum(WDEG)
    seg = _make_segsum(C)

    y0, y1 = _lin1(x, lin1_W, lin1_b.reshape(1, HID))

    deg2 = None
    for i in range(L):
        for g in range(G):
            y_in = y1 if g == 0 else y0
            res = y0 if g == 0 else y1
            first = deg2 is None
            z, p = _pre(y_in,
                        ln_g[i, g].reshape(1, C), ln_b[i, g].reshape(1, C),
                        proj_W[i, g], proj_b[i, g].reshape(1, C),
                        WDEG if first else C)
            if first:
                agg2 = seg_deg(p, src3, dst3, zdeg)
                deg2 = agg2
            else:
                agg2 = seg(p, src3, dst3, z64)
            y_new = _post(agg2, deg2, res, z,
                          ll_W[i, g], ll_b[i, g].reshape(1, C), lr_W[i, g])
            if g == 0:
                y0 = y_new
            else:
                y1 = y_new

    return _final(y0, y1, norm_g.reshape(1, HID), norm_b.reshape(1, HID),
                  lin2_W, lin2_b.reshape(1, OUT))


# speedup vs baseline: 11.9641x; 1.1294x over previous
import functools

import jax
import jax.numpy as jnp
from jax import lax
from jax.experimental import pallas as pl
from jax.experimental.pallas import tpu as pltpu
from jax.experimental.pallas import tpu_sc as plsc

N = 10000
E = 320000
D_IN = 128
HID = 128
OUT = 64
L = 3
G = 2
C = HID // G

NC = 2
NS = 16
NW = NC * NS
EPW = E // NW
CHUNK = 100
NCHUNK = EPW // CHUNK
NBUF = 5
NGROUP = NCHUNK // NBUF
N_PAD = 10240
RPT = N_PAD // NS
WDEG = 80


def _segsum_body(width, p_hbm, src_hbm, dst_hbm, z_hbm, agg_out,
                 sidx_v, didx_v, bufs, sems_g, sems_s, acc_sh):
    c = lax.axis_index("c")
    s = lax.axis_index("s")
    w = c * NS + s
    r0 = s * RPT

    pltpu.sync_copy(src_hbm.at[w], sidx_v)
    pltpu.sync_copy(dst_hbm.at[w], didx_v)
    pltpu.sync_copy(z_hbm, acc_sh.at[pl.ds(r0, RPT)])
    plsc.subcore_barrier()

    def group(g, carry):
        base = g * NBUF
        gh = []
        for b in range(NBUF):
            @pl.when(g > 0)
            def _drain(b=b):
                pltpu.make_async_copy(bufs[b], acc_sh.at[didx_v.at[0]],
                                      sems_s[b]).wait()
            gh.append(pltpu.async_copy(p_hbm.at[sidx_v.at[base + b]],
                                       bufs[b], sems_g[b]))
        for b in range(NBUF):
            gh[b].wait()
            pltpu.async_copy(bufs[b], acc_sh.at[didx_v.at[base + b]],
                             sems_s[b], add=True)
        return carry

    lax.fori_loop(0, NGROUP, group, 0)
    for b in range(NBUF):
        pltpu.make_async_copy(bufs[b], acc_sh.at[didx_v.at[0]],
                              sems_s[b]).wait()
    plsc.subcore_barrier()

    pltpu.sync_copy(acc_sh.at[pl.ds(r0, RPT)], agg_out.at[c, pl.ds(r0, RPT)])


def _make_segsum(width):
    mesh = plsc.VectorSubcoreMesh(core_axis_name="c", subcore_axis_name="s")

    def body(p_hbm, src_hbm, dst_hbm, z_hbm, agg_out,
             sidx_v, didx_v, *rest):
        bufs = rest[:NBUF]
        sems_g = rest[NBUF:2 * NBUF]
        sems_s = rest[2 * NBUF:3 * NBUF]
        acc_sh = rest[3 * NBUF]
        _segsum_body(width, p_hbm, src_hbm, dst_hbm, z_hbm, agg_out,
                     sidx_v, didx_v, bufs, sems_g, sems_s, acc_sh)

    scratch = [
        pltpu.VMEM((NCHUNK, CHUNK), jnp.int32),
        pltpu.VMEM((NCHUNK, CHUNK), jnp.int32),
    ]
    scratch += [pltpu.VMEM((CHUNK, width), jnp.float32) for _ in range(NBUF)]
    scratch += [pltpu.SemaphoreType.DMA for _ in range(2 * NBUF)]
    scratch.append(pltpu.VMEM_SHARED((N_PAD, width), jnp.float32))

    return pl.kernel(
        body,
        out_type=jax.ShapeDtypeStruct((NC, N_PAD, width), jnp.float32),
        mesh=mesh,
        scratch_types=scratch,
        compiler_params=pltpu.CompilerParams(use_tc_tiling_on_sc=False),
    )


RB = 1000
GRID = N // RB

def _ln(h, g, b):
    mu = jnp.mean(h, axis=-1, keepdims=True)
    var = jnp.mean((h - mu) ** 2, axis=-1, keepdims=True)
    return (h - mu) / jnp.sqrt(var + 1e-5) * g + b


def _lin1_body(x_ref, w_ref, b_ref, o0_ref, o1_ref):
    h = jnp.dot(x_ref[...], w_ref[...],
                preferred_element_type=jnp.float32) + b_ref[...]
    o0_ref[...] = h[:, :C]
    o1_ref[...] = h[:, C:]


def _lin1(x, w, b):
    return pl.pallas_call(
        _lin1_body,
        grid=(GRID,),
        in_specs=[
            pl.BlockSpec((RB, D_IN), lambda i: (i, 0)),
            pl.BlockSpec((D_IN, HID), lambda i: (0, 0)),
            pl.BlockSpec((1, HID), lambda i: (0, 0)),
        ],
        out_specs=[
            pl.BlockSpec((RB, C), lambda i: (i, 0)),
            pl.BlockSpec((RB, C), lambda i: (i, 0)),
        ],
        out_shape=[
            jax.ShapeDtypeStruct((N, C), jnp.float32),
            jax.ShapeDtypeStruct((N, C), jnp.float32),
        ],
    )(x, w, b)


def _pre_body(wout, y_ref, lg_ref, lb_ref, pw_ref, pb_ref, z_ref, p_ref):
    z = jax.nn.relu(_ln(y_ref[...], lg_ref[...], lb_ref[...]))
    z_ref[...] = z
    p = jax.nn.relu(
        jnp.dot(z, pw_ref[...], preferred_element_type=jnp.float32)
        + pb_ref[...])
    if wout > C:
        p = jnp.concatenate(
            [p, jnp.ones((p.shape[0], wout - C), jnp.float32)], axis=1)
    p_ref[...] = p


def _pre(y, lg, lb, pw, pb, wout):
    return pl.pallas_call(
        functools.partial(_pre_body, wout),
        grid=(GRID,),
        in_specs=[
            pl.BlockSpec((RB, C), lambda i: (i, 0)),
            pl.BlockSpec((1, C), lambda i: (0, 0)),
            pl.BlockSpec((1, C), lambda i: (0, 0)),
            pl.BlockSpec((C, C), lambda i: (0, 0)),
            pl.BlockSpec((1, C), lambda i: (0, 0)),
        ],
        out_specs=[
            pl.BlockSpec((RB, C), lambda i: (i, 0)),
            pl.BlockSpec((RB, wout), lambda i: (i, 0)),
        ],
        out_shape=[
            jax.ShapeDtypeStruct((N, C), jnp.float32),
            jax.ShapeDtypeStruct((N, wout), jnp.float32),
        ],
    )(y, lg, lb, pw, pb)


def _post_body(wa, a0_ref, a1_ref, d0_ref, d1_ref, res_ref, z_ref,
               lw_ref, lb_ref, rw_ref, y_ref):
    deg = jnp.clip(d0_ref[0][:, C:C + 1] + d1_ref[0][:, C:C + 1], 1.0, None)
    agg = (a0_ref[0][:, :C] + a1_ref[0][:, :C]) / deg
    y_ref[...] = (res_ref[...]
                  + jnp.dot(agg, lw_ref[...],
                            preferred_element_type=jnp.float32)
                  + lb_ref[...]
                  + jnp.dot(z_ref[...], rw_ref[...],
                            preferred_element_type=jnp.float32))


def _post(agg2, deg2, res, z, lw, lb, rw):
    wa = agg2.shape[-1]
    return pl.pallas_call(
        functools.partial(_post_body, wa),
        grid=(GRID,),
        in_specs=[
            pl.BlockSpec((1, RB, wa), lambda i: (0, i, 0)),
            pl.BlockSpec((1, RB, wa), lambda i: (1, i, 0)),
            pl.BlockSpec((1, RB, WDEG), lambda i: (0, i, 0)),
            pl.BlockSpec((1, RB, WDEG), lambda i: (1, i, 0)),
            pl.BlockSpec((RB, C), lambda i: (i, 0)),
            pl.BlockSpec((RB, C), lambda i: (i, 0)),
            pl.BlockSpec((C, C), lambda i: (0, 0)),
            pl.BlockSpec((1, C), lambda i: (0, 0)),
            pl.BlockSpec((C, C), lambda i: (0, 0)),
        ],
        out_specs=pl.BlockSpec((RB, C), lambda i: (i, 0)),
        out_shape=jax.ShapeDtypeStruct((N, C), jnp.float32),
    )(agg2, agg2, deg2, deg2, res, z, lw, lb, rw)


def _final_body(y0_ref, y1_ref, g_ref, b_ref, w_ref, b2_ref, o_ref):
    h = jnp.concatenate([y0_ref[...], y1_ref[...]], axis=1)
    h = jax.nn.relu(_ln(h, g_ref[...], b_ref[...]))
    o_ref[...] = jnp.dot(h, w_ref[...],
                         preferred_element_type=jnp.float32) + b2_ref[...]


def _final(y0, y1, g, b, w, b2):
    return pl.pallas_call(
        _final_body,
        grid=(GRID,),
        in_specs=[
            pl.BlockSpec((RB, C), lambda i: (i, 0)),
            pl.BlockSpec((RB, C), lambda i: (i, 0)),
            pl.BlockSpec((1, HID), lambda i: (0, 0)),
            pl.BlockSpec((1, HID), lambda i: (0, 0)),
            pl.BlockSpec((HID, OUT), lambda i: (0, 0)),
            pl.BlockSpec((1, OUT), lambda i: (0, 0)),
        ],
        out_specs=pl.BlockSpec((RB, OUT), lambda i: (i, 0)),
        out_shape=jax.ShapeDtypeStruct((N, OUT), jnp.float32),
    )(y0, y1, g, b, w, b2)


def kernel(x, edge_index, lin1_W, lin1_b, ln_g, ln_b, proj_W, proj_b,
           ll_W, ll_b, lr_W, norm_g, norm_b, lin2_W, lin2_b):
    src3 = edge_index[0].reshape(NW, NCHUNK, CHUNK)
    dst3 = edge_index[1].reshape(NW, NCHUNK, CHUNK)

    zdeg = jnp.zeros((RPT, WDEG), jnp.float32)
    z64 = jnp.zeros((RPT, C), jnp.float32)

    seg_deg = _make_segsum(WDEG)
    seg = _make_segsum(C)

    y0, y1 = _lin1(x, lin1_W, lin1_b.reshape(1, HID))

    deg2 = None
    for i in range(L):
        for g in range(G):
            y_in = y1 if g == 0 else y0
            res = y0 if g == 0 else y1
            first = deg2 is None
            z, p = _pre(y_in,
                        ln_g[i, g].reshape(1, C), ln_b[i, g].reshape(1, C),
                        proj_W[i, g], proj_b[i, g].reshape(1, C),
                        WDEG if first else C)
            if first:
                agg2 = seg_deg(p, src3, dst3, zdeg)
                deg2 = agg2
            else:
                agg2 = seg(p, src3, dst3, z64)
            y_new = _post(agg2, deg2, res, z,
                          ll_W[i, g], ll_b[i, g].reshape(1, C), lr_W[i, g])
            if g == 0:
                y0 = y_new
            else:
                y1 = y_new

    return _final(y0, y1, norm_g.reshape(1, HID), norm_b.reshape(1, HID),
                  lin2_W, lin2_b.reshape(1, OUT))


# speedup vs baseline: 12.6422x; 1.0567x over previous
import functools

import jax
import jax.numpy as jnp
from jax import lax
from jax.experimental import pallas as pl
from jax.experimental.pallas import tpu as pltpu
from jax.experimental.pallas import tpu_sc as plsc

N = 10000
E = 320000
D_IN = 128
HID = 128
OUT = 64
L = 3
G = 2
C = HID // G

NC = 2
NS = 16
NW = NC * NS
EPW = E // NW
CHUNK = 100
NCHUNK = EPW // CHUNK
NBUF = 5
NGROUP = NCHUNK // NBUF
N_PAD = 10240
RPT = N_PAD // NS
WDEG = 80


def _segsum_body(width, p_hbm, src_hbm, dst_hbm, z_hbm, agg_out,
                 sidx_v, didx_v, bufs, sems_g, sems_s, acc_sh):
    c = lax.axis_index("c")
    s = lax.axis_index("s")
    w = c * NS + s
    r0 = s * RPT

    pltpu.sync_copy(src_hbm.at[w], sidx_v)
    pltpu.sync_copy(dst_hbm.at[w], didx_v)
    pltpu.sync_copy(z_hbm, acc_sh.at[pl.ds(r0, RPT)])
    plsc.subcore_barrier()

    def group(g, carry):
        base = g * NBUF
        gh = []
        for b in range(NBUF):
            @pl.when(g > 0)
            def _drain(b=b):
                pltpu.make_async_copy(bufs[b], acc_sh.at[didx_v.at[0]],
                                      sems_s[b]).wait()
            gh.append(pltpu.async_copy(p_hbm.at[sidx_v.at[base + b]],
                                       bufs[b], sems_g[b]))
        for b in range(NBUF):
            gh[b].wait()
            pltpu.async_copy(bufs[b], acc_sh.at[didx_v.at[base + b]],
                             sems_s[b], add=True)
        return carry

    lax.fori_loop(0, NGROUP, group, 0)
    for b in range(NBUF):
        pltpu.make_async_copy(bufs[b], acc_sh.at[didx_v.at[0]],
                              sems_s[b]).wait()
    plsc.subcore_barrier()

    pltpu.sync_copy(acc_sh.at[pl.ds(r0, RPT)], agg_out.at[c, pl.ds(r0, RPT)])


def _make_segsum(width):
    mesh = plsc.VectorSubcoreMesh(core_axis_name="c", subcore_axis_name="s")

    def body(p_hbm, src_hbm, dst_hbm, z_hbm, agg_out,
             sidx_v, didx_v, *rest):
        bufs = rest[:NBUF]
        sems_g = rest[NBUF:2 * NBUF]
        sems_s = rest[2 * NBUF:3 * NBUF]
        acc_sh = rest[3 * NBUF]
        _segsum_body(width, p_hbm, src_hbm, dst_hbm, z_hbm, agg_out,
                     sidx_v, didx_v, bufs, sems_g, sems_s, acc_sh)

    scratch = [
        pltpu.VMEM((NCHUNK, CHUNK), jnp.int32),
        pltpu.VMEM((NCHUNK, CHUNK), jnp.int32),
    ]
    scratch += [pltpu.VMEM((CHUNK, width), jnp.float32) for _ in range(NBUF)]
    scratch += [pltpu.SemaphoreType.DMA for _ in range(2 * NBUF)]
    scratch.append(pltpu.VMEM_SHARED((N_PAD, width), jnp.float32))

    return pl.kernel(
        body,
        out_type=jax.ShapeDtypeStruct((NC, N_PAD, width), jnp.float32),
        mesh=mesh,
        scratch_types=scratch,
        compiler_params=pltpu.CompilerParams(use_tc_tiling_on_sc=False),
    )


RB = 1000
GRID = N // RB

def _ln(h, g, b):
    mu = jnp.mean(h, axis=-1, keepdims=True)
    var = jnp.mean((h - mu) ** 2, axis=-1, keepdims=True)
    return (h - mu) / jnp.sqrt(var + 1e-5) * g + b


def _head_body(x_ref, w_ref, b_ref, lg_ref, lb_ref, pw_ref, pb_ref,
               y0_ref, y1_ref, z_ref, p_ref):
    h = jnp.dot(x_ref[...], w_ref[...],
                preferred_element_type=jnp.float32) + b_ref[...]
    y0 = h[:, :C]
    y1 = h[:, C:]
    y0_ref[...] = y0
    y1_ref[...] = y1
    z = jax.nn.relu(_ln(y1, lg_ref[...], lb_ref[...]))
    z_ref[...] = z
    p = jax.nn.relu(
        jnp.dot(z, pw_ref[...], preferred_element_type=jnp.float32)
        + pb_ref[...])
    p_ref[...] = jnp.concatenate(
        [p, jnp.ones((p.shape[0], WDEG - C), jnp.float32)], axis=1)


def _head(x, w, b, lg, lb, pw, pb):
    return pl.pallas_call(
        _head_body,
        grid=(GRID,),
        in_specs=[
            pl.BlockSpec((RB, D_IN), lambda i: (i, 0)),
            pl.BlockSpec((D_IN, HID), lambda i: (0, 0)),
            pl.BlockSpec((1, HID), lambda i: (0, 0)),
            pl.BlockSpec((1, C), lambda i: (0, 0)),
            pl.BlockSpec((1, C), lambda i: (0, 0)),
            pl.BlockSpec((C, C), lambda i: (0, 0)),
            pl.BlockSpec((1, C), lambda i: (0, 0)),
        ],
        out_specs=[
            pl.BlockSpec((RB, C), lambda i: (i, 0)),
            pl.BlockSpec((RB, C), lambda i: (i, 0)),
            pl.BlockSpec((RB, C), lambda i: (i, 0)),
            pl.BlockSpec((RB, WDEG), lambda i: (i, 0)),
        ],
        out_shape=[
            jax.ShapeDtypeStruct((N, C), jnp.float32),
            jax.ShapeDtypeStruct((N, C), jnp.float32),
            jax.ShapeDtypeStruct((N, C), jnp.float32),
            jax.ShapeDtypeStruct((N, WDEG), jnp.float32),
        ],
    )(x, w, b, lg, lb, pw, pb)


def _step_first_body(a0_ref, a1_ref, res_ref, z_ref, lw_ref, lb_ref, rw_ref,
                     lg2_ref, lb2_ref, pw2_ref, pb2_ref,
                     y_ref, z2_ref, p2_ref, dinv_ref):
    a0 = a0_ref[0]
    a1 = a1_ref[0]
    dinv = 1.0 / jnp.clip(a0[:, C:C + 1] + a1[:, C:C + 1], 1.0, None)
    dinv_ref[...] = jnp.broadcast_to(dinv, dinv_ref.shape)
    agg = (a0[:, :C] + a1[:, :C]) * dinv
    y = (res_ref[...]
         + jnp.dot(agg, lw_ref[...], preferred_element_type=jnp.float32)
         + lb_ref[...]
         + jnp.dot(z_ref[...], rw_ref[...],
                   preferred_element_type=jnp.float32))
    y_ref[...] = y
    z2 = jax.nn.relu(_ln(y, lg2_ref[...], lb2_ref[...]))
    z2_ref[...] = z2
    p2_ref[...] = jax.nn.relu(
        jnp.dot(z2, pw2_ref[...], preferred_element_type=jnp.float32)
        + pb2_ref[...])


def _step_first(agg2, res, z, lw, lb, rw, lg2, lb2, pw2, pb2):
    return pl.pallas_call(
        _step_first_body,
        grid=(GRID,),
        in_specs=[
            pl.BlockSpec((1, RB, WDEG), lambda i: (0, i, 0)),
            pl.BlockSpec((1, RB, WDEG), lambda i: (1, i, 0)),
            pl.BlockSpec((RB, C), lambda i: (i, 0)),
            pl.BlockSpec((RB, C), lambda i: (i, 0)),
            pl.BlockSpec((C, C), lambda i: (0, 0)),
            pl.BlockSpec((1, C), lambda i: (0, 0)),
            pl.BlockSpec((C, C), lambda i: (0, 0)),
            pl.BlockSpec((1, C), lambda i: (0, 0)),
            pl.BlockSpec((1, C), lambda i: (0, 0)),
            pl.BlockSpec((C, C), lambda i: (0, 0)),
            pl.BlockSpec((1, C), lambda i: (0, 0)),
        ],
        out_specs=[
            pl.BlockSpec((RB, C), lambda i: (i, 0)),
            pl.BlockSpec((RB, C), lambda i: (i, 0)),
            pl.BlockSpec((RB, C), lambda i: (i, 0)),
            pl.BlockSpec((RB, 8), lambda i: (i, 0)),
        ],
        out_shape=[
            jax.ShapeDtypeStruct((N, C), jnp.float32),
            jax.ShapeDtypeStruct((N, C), jnp.float32),
            jax.ShapeDtypeStruct((N, C), jnp.float32),
            jax.ShapeDtypeStruct((N, 8), jnp.float32),
        ],
    )(agg2, agg2, res, z, lw, lb, rw, lg2, lb2, pw2, pb2)


def _step_body(a0_ref, a1_ref, dinv_ref, res_ref, z_ref,
               lw_ref, lb_ref, rw_ref, lg2_ref, lb2_ref, pw2_ref, pb2_ref,
               y_ref, z2_ref, p2_ref):
    agg = (a0_ref[0] + a1_ref[0]) * dinv_ref[:, :1]
    y = (res_ref[...]
         + jnp.dot(agg, lw_ref[...], preferred_element_type=jnp.float32)
         + lb_ref[...]
         + jnp.dot(z_ref[...], rw_ref[...],
                   preferred_element_type=jnp.float32))
    y_ref[...] = y
    z2 = jax.nn.relu(_ln(y, lg2_ref[...], lb2_ref[...]))
    z2_ref[...] = z2
    p2_ref[...] = jax.nn.relu(
        jnp.dot(z2, pw2_ref[...], preferred_element_type=jnp.float32)
        + pb2_ref[...])


def _step(agg2, dinv, res, z, lw, lb, rw, lg2, lb2, pw2, pb2):
    return pl.pallas_call(
        _step_body,
        grid=(GRID,),
        in_specs=[
            pl.BlockSpec((1, RB, C), lambda i: (0, i, 0)),
            pl.BlockSpec((1, RB, C), lambda i: (1, i, 0)),
            pl.BlockSpec((RB, 8), lambda i: (i, 0)),
            pl.BlockSpec((RB, C), lambda i: (i, 0)),
            pl.BlockSpec((RB, C), lambda i: (i, 0)),
            pl.BlockSpec((C, C), lambda i: (0, 0)),
            pl.BlockSpec((1, C), lambda i: (0, 0)),
            pl.BlockSpec((C, C), lambda i: (0, 0)),
            pl.BlockSpec((1, C), lambda i: (0, 0)),
            pl.BlockSpec((1, C), lambda i: (0, 0)),
            pl.BlockSpec((C, C), lambda i: (0, 0)),
            pl.BlockSpec((1, C), lambda i: (0, 0)),
        ],
        out_specs=[
            pl.BlockSpec((RB, C), lambda i: (i, 0)),
            pl.BlockSpec((RB, C), lambda i: (i, 0)),
            pl.BlockSpec((RB, C), lambda i: (i, 0)),
        ],
        out_shape=[
            jax.ShapeDtypeStruct((N, C), jnp.float32),
            jax.ShapeDtypeStruct((N, C), jnp.float32),
            jax.ShapeDtypeStruct((N, C), jnp.float32),
        ],
    )(agg2, agg2, dinv, res, z, lw, lb, rw, lg2, lb2, pw2, pb2)


def _tail_body(a0_ref, a1_ref, dinv_ref, res_ref, z_ref,
               lw_ref, lb_ref, rw_ref, y0_ref, g_ref, b_ref, w2_ref, b2_ref,
               o_ref):
    agg = (a0_ref[0] + a1_ref[0]) * dinv_ref[:, :1]
    y1 = (res_ref[...]
          + jnp.dot(agg, lw_ref[...], preferred_element_type=jnp.float32)
          + lb_ref[...]
          + jnp.dot(z_ref[...], rw_ref[...],
                    preferred_element_type=jnp.float32))
    h = jnp.concatenate([y0_ref[...], y1], axis=1)
    h = jax.nn.relu(_ln(h, g_ref[...], b_ref[...]))
    o_ref[...] = jnp.dot(h, w2_ref[...],
                         preferred_element_type=jnp.float32) + b2_ref[...]


def _tail(agg2, dinv, res, z, lw, lb, rw, y0, g, b, w2, b2):
    return pl.pallas_call(
        _tail_body,
        grid=(GRID,),
        in_specs=[
            pl.BlockSpec((1, RB, C), lambda i: (0, i, 0)),
            pl.BlockSpec((1, RB, C), lambda i: (1, i, 0)),
            pl.BlockSpec((RB, 8), lambda i: (i, 0)),
            pl.BlockSpec((RB, C), lambda i: (i, 0)),
            pl.BlockSpec((RB, C), lambda i: (i, 0)),
            pl.BlockSpec((C, C), lambda i: (0, 0)),
            pl.BlockSpec((1, C), lambda i: (0, 0)),
            pl.BlockSpec((C, C), lambda i: (0, 0)),
            pl.BlockSpec((RB, C), lambda i: (i, 0)),
            pl.BlockSpec((1, HID), lambda i: (0, 0)),
            pl.BlockSpec((1, HID), lambda i: (0, 0)),
            pl.BlockSpec((HID, OUT), lambda i: (0, 0)),
            pl.BlockSpec((1, OUT), lambda i: (0, 0)),
        ],
        out_specs=pl.BlockSpec((RB, OUT), lambda i: (i, 0)),
        out_shape=jax.ShapeDtypeStruct((N, OUT), jnp.float32),
    )(agg2, agg2, dinv, res, z, lw, lb, rw, y0, g, b, w2, b2)


def kernel(x, edge_index, lin1_W, lin1_b, ln_g, ln_b, proj_W, proj_b,
           ll_W, ll_b, lr_W, norm_g, norm_b, lin2_W, lin2_b):
    src3 = edge_index[0].reshape(NW, NCHUNK, CHUNK)
    dst3 = edge_index[1].reshape(NW, NCHUNK, CHUNK)

    zdeg = jnp.zeros((RPT, WDEG), jnp.float32)
    z64 = jnp.zeros((RPT, C), jnp.float32)

    seg_deg = _make_segsum(WDEG)
    seg = _make_segsum(C)

    lg = lambda i, g: ln_g[i, g].reshape(1, C)
    lb_ = lambda i, g: ln_b[i, g].reshape(1, C)
    pb = lambda i, g: proj_b[i, g].reshape(1, C)
    llb = lambda i, g: ll_b[i, g].reshape(1, C)

    y0, y1, z, p = _head(x, lin1_W, lin1_b.reshape(1, HID),
                         lg(0, 0), lb_(0, 0), proj_W[0, 0], pb(0, 0))

    blocks = [(i, g) for i in range(L) for g in range(G)]
    dinv = None
    for k, (i, g) in enumerate(blocks[:-1]):
        ni, ng = blocks[k + 1]
        res = y0 if g == 0 else y1
        if k == 0:
            agg2 = seg_deg(p, src3, dst3, zdeg)
            y_new, z, p, dinv = _step_first(
                agg2, res, z, ll_W[i, g], llb(i, g), lr_W[i, g],
                lg(ni, ng), lb_(ni, ng), proj_W[ni, ng], pb(ni, ng))
        else:
            agg2 = seg(p, src3, dst3, z64)
            y_new, z, p = _step(
                agg2, dinv, res, z, ll_W[i, g], llb(i, g), lr_W[i, g],
                lg(ni, ng), lb_(ni, ng), proj_W[ni, ng], pb(ni, ng))
        if g == 0:
            y0 = y_new
        else:
            y1 = y_new

    i, g = blocks[-1]
    agg2 = seg(p, src3, dst3, z64)
    return _tail(agg2, dinv, y1, z, ll_W[i, g], llb(i, g), lr_W[i, g], y0,
                 norm_g.reshape(1, HID), norm_b.reshape(1, HID),
                 lin2_W, lin2_b.reshape(1, OUT))
